# baseline (device time: 40862 ns/iter reference)
import functools

import jax
import jax.numpy as jnp
from jax import lax
from jax.experimental import pallas as pl
from jax.experimental.pallas import tpu as pltpu

N_DEV = 4


def kernel(x, Wq, K_ext, V_ext, Wo):
    B, Sq, Dm = x.shape
    _, Skv_loc, Hq, Dh = K_ext.shape
    Hl = Wq.shape[1] // Dh
    Dout = Wo.shape[1]
    Cb = Hl * Dh

    K2 = K_ext.reshape(B, Skv_loc, Hq * Dh)
    V2 = V_ext.reshape(B, Skv_loc, Hq * Dh)

    def body(x_ref, wq_ref, k_ref, v_ref, wo_ref, out_ref,
             kv_local, kv_recv, out_send, out_recv,
             kv_send_sems, kv_recv_sems, o_send_sems, o_recv_sems):
        my = lax.axis_index("i")

        barrier_sem = pltpu.get_barrier_semaphore()
        for p in range(N_DEV):
            @pl.when(my != p)
            def _(p=p):
                pl.semaphore_signal(
                    barrier_sem, inc=1,
                    device_id=(p,), device_id_type=pl.DeviceIdType.MESH,
                )
        pl.semaphore_wait(barrier_sem, N_DEV - 1)

        k_all = k_ref[...].astype(jnp.bfloat16)
        v_all = v_ref[...].astype(jnp.bfloat16)
        for c in range(N_DEV):
            kv_local[c, 0] = k_all[:, :, c * Cb:(c + 1) * Cb]
            kv_local[c, 1] = v_all[:, :, c * Cb:(c + 1) * Cb]

        for c in range(N_DEV):
            @pl.when(my != c)
            def _(c=c):
                rdma = pltpu.make_async_remote_copy(
                    src_ref=kv_local.at[c],
                    dst_ref=kv_recv.at[my],
                    send_sem=kv_send_sems.at[c],
                    recv_sem=kv_recv_sems.at[my],
                    device_id=(c,),
                    device_id_type=pl.DeviceIdType.MESH,
                )
                rdma.start()

            @pl.when(my == c)
            def _(c=c):
                kv_recv[c, 0] = k_all[:, :, c * Cb:(c + 1) * Cb]
                kv_recv[c, 1] = v_all[:, :, c * Cb:(c + 1) * Cb]

        wq = wq_ref[...].astype(jnp.bfloat16)
        x_all = x_ref[...].astype(jnp.bfloat16)
        q = []
        for b in range(B):
            qb = lax.dot_general(
                x_all[b], wq, (((1,), (0,)), ((), ())),
                preferred_element_type=jnp.float32,
            )
            q.append(qb.astype(jnp.bfloat16))

        for s in range(N_DEV):
            @pl.when(my != s)
            def _(s=s):
                desc = pltpu.make_async_remote_copy(
                    src_ref=kv_local.at[s],
                    dst_ref=kv_recv.at[s],
                    send_sem=kv_send_sems.at[s],
                    recv_sem=kv_recv_sems.at[s],
                    device_id=(s,),
                    device_id_type=pl.DeviceIdType.MESH,
                )
                desc.wait_recv()

        qi = lax.broadcasted_iota(jnp.int32, (Sq, Skv_loc), 0)
        kj0 = lax.broadcasted_iota(jnp.int32, (Sq, Skv_loc), 1)
        wo = wo_ref[...].astype(jnp.bfloat16)
        partials = []
        for b in range(B):
            ctx_list = []
            for h in range(Hl):
                qbh = q[b][:, h * Dh:(h + 1) * Dh]
                scores = []
                for s in range(N_DEV):
                    ks = kv_recv[s, 0, b, :, h * Dh:(h + 1) * Dh]
                    sc = lax.dot_general(
                        qbh, ks, (((1,), (1,)), ((), ())),
                        preferred_element_type=jnp.float32,
                    ) * 0.125
                    kj = kj0 + s * Skv_loc
                    mask = (jnp.abs(qi - kj) <= 128) | (kj < 32) | (qi < 32)
                    scores.append(jnp.where(mask, sc, -1e9))
                m = functools.reduce(jnp.maximum, scores).max(axis=1, keepdims=True)
                ws = [jnp.exp(sc - m) for sc in scores]
                denom = functools.reduce(
                    jnp.add, [w.sum(axis=1, keepdims=True) for w in ws]
                )
                ctx_un = functools.reduce(jnp.add, [
                    lax.dot_general(
                        ws[s].astype(jnp.bfloat16),
                        kv_recv[s, 1, b, :, h * Dh:(h + 1) * Dh],
                        (((1,), (0,)), ((), ())),
                        preferred_element_type=jnp.float32,
                    )
                    for s in range(N_DEV)
                ])
                ctx_list.append(ctx_un / denom)
            ctx_b = jnp.concatenate(ctx_list, axis=1).astype(jnp.bfloat16)
            pb = lax.dot_general(
                ctx_b, wo, (((1,), (0,)), ((), ())),
                preferred_element_type=jnp.float32,
            )
            partials.append(pb)
            out_send[b] = pb.astype(jnp.bfloat16)

        for c in range(N_DEV):
            @pl.when(my != c)
            def _(c=c):
                rdma = pltpu.make_async_remote_copy(
                    src_ref=out_send,
                    dst_ref=out_recv.at[my],
                    send_sem=o_send_sems.at[c],
                    recv_sem=o_recv_sems.at[my],
                    device_id=(c,),
                    device_id_type=pl.DeviceIdType.MESH,
                )
                rdma.start()

            @pl.when(my == c)
            def _(c=c):
                for b in range(B):
                    out_recv[c, b] = partials[b].astype(jnp.bfloat16)

        for c in range(N_DEV):
            @pl.when(my != c)
            def _(c=c):
                desc = pltpu.make_async_remote_copy(
                    src_ref=kv_local.at[c],
                    dst_ref=kv_recv.at[c],
                    send_sem=kv_send_sems.at[c],
                    recv_sem=kv_recv_sems.at[c],
                    device_id=(c,),
                    device_id_type=pl.DeviceIdType.MESH,
                )
                desc.wait_send()

        for s in range(N_DEV):
            @pl.when(my != s)
            def _(s=s):
                desc = pltpu.make_async_remote_copy(
                    src_ref=out_send,
                    dst_ref=out_recv.at[s],
                    send_sem=o_send_sems.at[s],
                    recv_sem=o_recv_sems.at[s],
                    device_id=(s,),
                    device_id_type=pl.DeviceIdType.MESH,
                )
                desc.wait_recv()

        acc = functools.reduce(jnp.add, [
            out_recv[s].astype(jnp.float32) for s in range(N_DEV)
        ])
        out_ref[...] = acc

        for c in range(N_DEV):
            @pl.when(my != c)
            def _(c=c):
                desc = pltpu.make_async_remote_copy(
                    src_ref=out_send,
                    dst_ref=out_recv.at[c],
                    send_sem=o_send_sems.at[c],
                    recv_sem=o_recv_sems.at[c],
                    device_id=(c,),
                    device_id_type=pl.DeviceIdType.MESH,
                )
                desc.wait_send()

    return pl.pallas_call(
        body,
        out_shape=jax.ShapeDtypeStruct((B, Sq, Dout), jnp.float32),
        in_specs=[pl.BlockSpec(memory_space=pltpu.VMEM)] * 5,
        out_specs=pl.BlockSpec(memory_space=pltpu.VMEM),
        scratch_shapes=[
            pltpu.VMEM((N_DEV, 2, B, Skv_loc, Cb), jnp.bfloat16),
            pltpu.VMEM((N_DEV, 2, B, Skv_loc, Cb), jnp.bfloat16),
            pltpu.VMEM((B, Sq, Dout), jnp.bfloat16),
            pltpu.VMEM((N_DEV, B, Sq, Dout), jnp.bfloat16),
            pltpu.SemaphoreType.DMA((N_DEV,)),
            pltpu.SemaphoreType.DMA((N_DEV,)),
            pltpu.SemaphoreType.DMA((N_DEV,)),
            pltpu.SemaphoreType.DMA((N_DEV,)),
        ],
        compiler_params=pltpu.CompilerParams(collective_id=0),
    )(x, Wq, K2, V2, Wo)


# device time: 39105 ns/iter; 1.0449x vs baseline; 1.0449x over previous
import functools

import jax
import jax.numpy as jnp
from jax import lax
from jax.experimental import pallas as pl
from jax.experimental.pallas import tpu as pltpu

N_DEV = 4

ROW_GROUPS = ((0, 32, 1024), (32, 128, 256), (128, 256, 512))


def kernel(x, Wq, K_ext, V_ext, Wo):
    B, Sq, Dm = x.shape
    _, Skv_loc, Hq, Dh = K_ext.shape
    Hl = Wq.shape[1] // Dh
    Dout = Wo.shape[1]
    Cb = Hl * Dh

    K2 = K_ext.reshape(B, Skv_loc, Hq * Dh)
    V2 = V_ext.reshape(B, Skv_loc, Hq * Dh)

    def body(x_ref, wq_ref, k_ref, v_ref, wo_ref, out_ref,
             kv_local, kv_recv, out_send, out_recv,
             kv_send_sems, kv_recv_sems, o_send_sems, o_recv_sems, loc_sem):
        my = lax.axis_index("i")

        barrier_sem = pltpu.get_barrier_semaphore()
        for p in range(N_DEV):
            @pl.when(my != p)
            def _(p=p):
                pl.semaphore_signal(
                    barrier_sem, inc=1,
                    device_id=(p,), device_id_type=pl.DeviceIdType.MESH,
                )
        pl.semaphore_wait(barrier_sem, N_DEV - 1)

        k_all = k_ref[...].astype(jnp.bfloat16)
        v_all = v_ref[...].astype(jnp.bfloat16)
        for c in range(N_DEV):
            kv_local[c, 0] = k_all[:, :, c * Cb:(c + 1) * Cb]
            kv_local[c, 1] = v_all[:, :, c * Cb:(c + 1) * Cb]

        for c in range(N_DEV):
            @pl.when(my != c)
            def _(c=c):
                rdma = pltpu.make_async_remote_copy(
                    src_ref=kv_local.at[c],
                    dst_ref=kv_recv.at[:, :, pl.ds(my * Skv_loc, Skv_loc), :],
                    send_sem=kv_send_sems.at[c],
                    recv_sem=kv_recv_sems.at[my],
                    device_id=(c,),
                    device_id_type=pl.DeviceIdType.MESH,
                )
                rdma.start()
        loc_copy = pltpu.make_async_copy(
            kv_local.at[my],
            kv_recv.at[:, :, pl.ds(my * Skv_loc, Skv_loc), :],
            loc_sem.at[0],
        )
        loc_copy.start()

        wq = wq_ref[...].astype(jnp.bfloat16)
        x_all = x_ref[...].astype(jnp.bfloat16)
        qs = []
        for b in range(B):
            qb = lax.dot_general(
                x_all[b], wq, (((1,), (0,)), ((), ())),
                preferred_element_type=jnp.float32,
            )
            qs.append((qb * 0.125).astype(jnp.bfloat16))

        biases = []
        for (r0, r1, nc) in ROW_GROUPS:
            qi = r0 + lax.broadcasted_iota(jnp.int32, (r1 - r0, nc), 0)
            kj = lax.broadcasted_iota(jnp.int32, (r1 - r0, nc), 1)
            mask = (jnp.abs(qi - kj) <= 128) | (kj < 32) | (qi < 32)
            biases.append(jnp.where(mask, 0.0, -1e9).astype(jnp.float32))

        loc_copy.wait()
        for s in range(N_DEV):
            @pl.when(my != s)
            def _(s=s):
                desc = pltpu.make_async_remote_copy(
                    src_ref=kv_local.at[s],
                    dst_ref=kv_recv.at[:, :, pl.ds(s * Skv_loc, Skv_loc), :],
                    send_sem=kv_send_sems.at[s],
                    recv_sem=kv_recv_sems.at[s],
                    device_id=(s,),
                    device_id_type=pl.DeviceIdType.MESH,
                )
                desc.wait_recv()

        wo = wo_ref[...].astype(jnp.bfloat16)
        for b in range(B):
            group_ctx = [[] for _ in ROW_GROUPS]
            for h in range(Hl):
                qbh = qs[b][:, h * Dh:(h + 1) * Dh]
                for g, (r0, r1, nc) in enumerate(ROW_GROUPS):
                    kg = kv_recv[0, b, 0:nc, h * Dh:(h + 1) * Dh]
                    sc = lax.dot_general(
                        qbh[r0:r1], kg, (((1,), (1,)), ((), ())),
                        preferred_element_type=jnp.float32,
                    ) + biases[g]
                    m = sc.max(axis=1, keepdims=True)
                    w = jnp.exp(sc - m)
                    denom = w.sum(axis=1, keepdims=True)
                    vg = kv_recv[1, b, 0:nc, h * Dh:(h + 1) * Dh]
                    ctx = lax.dot_general(
                        w.astype(jnp.bfloat16), vg, (((1,), (0,)), ((), ())),
                        preferred_element_type=jnp.float32,
                    ) / denom
                    group_ctx[g].append(ctx)
            ctx_b = jnp.concatenate(
                [jnp.concatenate(hs, axis=1) for hs in group_ctx], axis=0
            ).astype(jnp.bfloat16)
            pb = lax.dot_general(
                ctx_b, wo, (((1,), (0,)), ((), ())),
                preferred_element_type=jnp.float32,
            )
            out_send[b] = pb.astype(jnp.bfloat16)

            for c in range(N_DEV):
                @pl.when(my != c)
                def _(c=c, b=b):
                    rdma = pltpu.make_async_remote_copy(
                        src_ref=out_send.at[b],
                        dst_ref=out_recv.at[my, b],
                        send_sem=o_send_sems.at[c, b],
                        recv_sem=o_recv_sems.at[my, b],
                        device_id=(c,),
                        device_id_type=pl.DeviceIdType.MESH,
                    )
                    rdma.start()

                @pl.when(my == c)
                def _(c=c, b=b):
                    out_recv[c, b] = pb.astype(jnp.bfloat16)

        for c in range(N_DEV):
            @pl.when(my != c)
            def _(c=c):
                desc = pltpu.make_async_remote_copy(
                    src_ref=kv_local.at[c],
                    dst_ref=kv_recv.at[:, :, pl.ds(c * Skv_loc, Skv_loc), :],
                    send_sem=kv_send_sems.at[c],
                    recv_sem=kv_recv_sems.at[c],
                    device_id=(c,),
                    device_id_type=pl.DeviceIdType.MESH,
                )
                desc.wait_send()

        for b in range(B):
            for s in range(N_DEV):
                @pl.when(my != s)
                def _(s=s, b=b):
                    desc = pltpu.make_async_remote_copy(
                        src_ref=out_send.at[b],
                        dst_ref=out_recv.at[s, b],
                        send_sem=o_send_sems.at[s, b],
                        recv_sem=o_recv_sems.at[s, b],
                        device_id=(s,),
                        device_id_type=pl.DeviceIdType.MESH,
                    )
                    desc.wait_recv()
            out_ref[b] = functools.reduce(jnp.add, [
                out_recv[s, b].astype(jnp.float32) for s in range(N_DEV)
            ])

        for b in range(B):
            for c in range(N_DEV):
                @pl.when(my != c)
                def _(c=c, b=b):
                    desc = pltpu.make_async_remote_copy(
                        src_ref=out_send.at[b],
                        dst_ref=out_recv.at[c, b],
                        send_sem=o_send_sems.at[c, b],
                        recv_sem=o_recv_sems.at[c, b],
                        device_id=(c,),
                        device_id_type=pl.DeviceIdType.MESH,
                    )
                    desc.wait_send()

    return pl.pallas_call(
        body,
        out_shape=jax.ShapeDtypeStruct((B, Sq, Dout), jnp.float32),
        in_specs=[pl.BlockSpec(memory_space=pltpu.VMEM)] * 5,
        out_specs=pl.BlockSpec(memory_space=pltpu.VMEM),
        scratch_shapes=[
            pltpu.VMEM((N_DEV, 2, B, Skv_loc, Cb), jnp.bfloat16),
            pltpu.VMEM((2, B, N_DEV * Skv_loc, Cb), jnp.bfloat16),
            pltpu.VMEM((B, Sq, Dout), jnp.bfloat16),
            pltpu.VMEM((N_DEV, B, Sq, Dout), jnp.bfloat16),
            pltpu.SemaphoreType.DMA((N_DEV,)),
            pltpu.SemaphoreType.DMA((N_DEV,)),
            pltpu.SemaphoreType.DMA((N_DEV, B)),
            pltpu.SemaphoreType.DMA((N_DEV, B)),
            pltpu.SemaphoreType.DMA((1,)),
        ],
        compiler_params=pltpu.CompilerParams(collective_id=0),
    )(x, Wq, K2, V2, Wo)


# device time: 29545 ns/iter; 1.3830x vs baseline; 1.3236x over previous
import functools

import jax
import jax.numpy as jnp
from jax import lax
from jax.experimental import pallas as pl
from jax.experimental.pallas import tpu as pltpu

N_DEV = 4

ROW_GROUPS = ((0, 32, 1024), (32, 128, 256), (128, 256, 512))


def kernel(x, Wq, K_ext, V_ext, Wo):
    B, Sq, Dm = x.shape
    _, Skv_loc, Hq, Dh = K_ext.shape
    Hl = Wq.shape[1] // Dh
    Dout = Wo.shape[1]
    Cb = Hl * Dh
    Qr = Sq // N_DEV

    K2 = K_ext.reshape(B, Skv_loc, Hq * Dh).astype(jnp.bfloat16)
    V2 = V_ext.reshape(B, Skv_loc, Hq * Dh).astype(jnp.bfloat16)

    def body(x_ref, wq_ref, k_ref, v_ref, wo_ref, out_ref,
             kv_recv, out_send, rs_recv, ag_send, ag_recv,
             kv_send_sems, kv_recv_sems, v_send_sems, v_recv_sems,
             rs_send_sems, rs_recv_sems,
             ag_send_sems, ag_recv_sems, loc_sem):
        my = lax.axis_index("i")

        barrier_sem = pltpu.get_barrier_semaphore()
        for p in range(N_DEV):
            @pl.when(my != p)
            def _(p=p):
                pl.semaphore_signal(
                    barrier_sem, inc=1,
                    device_id=(p,), device_id_type=pl.DeviceIdType.MESH,
                )
        pl.semaphore_wait(barrier_sem, N_DEV - 1)

        for b in range(B):
            for kv in range(2):
                for c in range(N_DEV):
                    @pl.when(my != c)
                    def _(c=c, b=b, kv=kv):
                        sems = (kv_send_sems, kv_recv_sems) if kv == 0 else (
                            v_send_sems, v_recv_sems)
                        rdma = pltpu.make_async_remote_copy(
                            src_ref=(k_ref if kv == 0 else v_ref).at[b, :, c * Cb:(c + 1) * Cb],
                            dst_ref=kv_recv.at[kv, b, pl.ds(my * Skv_loc, Skv_loc), :],
                            send_sem=sems[0].at[c, b],
                            recv_sem=sems[1].at[my, b],
                            device_id=(c,),
                            device_id_type=pl.DeviceIdType.MESH,
                        )
                        rdma.start()
        for c in range(N_DEV):
            @pl.when(my == c)
            def _(c=c):
                pltpu.make_async_copy(
                    k_ref.at[:, :, c * Cb:(c + 1) * Cb],
                    kv_recv.at[0, :, c * Skv_loc:(c + 1) * Skv_loc, :],
                    loc_sem.at[0],
                ).start()
                pltpu.make_async_copy(
                    v_ref.at[:, :, c * Cb:(c + 1) * Cb],
                    kv_recv.at[1, :, c * Skv_loc:(c + 1) * Skv_loc, :],
                    loc_sem.at[1],
                ).start()

        wq = wq_ref[...].astype(jnp.bfloat16)
        x_all = x_ref[...].astype(jnp.bfloat16)
        qs = []
        for b in range(B):
            qb = lax.dot_general(
                x_all[b], wq, (((1,), (0,)), ((), ())),
                preferred_element_type=jnp.float32,
            )
            qs.append((qb * 0.125).astype(jnp.bfloat16))

        biases = []
        for (r0, r1, nc) in ROW_GROUPS:
            qi = r0 + lax.broadcasted_iota(jnp.int32, (r1 - r0, nc), 0)
            kj = lax.broadcasted_iota(jnp.int32, (r1 - r0, nc), 1)
            mask = (jnp.abs(qi - kj) <= 128) | (kj < 32) | (qi < 32)
            biases.append(jnp.where(mask, 0.0, -1e9).astype(jnp.float32))

        for c in range(N_DEV):
            @pl.when(my == c)
            def _(c=c):
                pltpu.make_async_copy(
                    k_ref.at[:, :, c * Cb:(c + 1) * Cb],
                    kv_recv.at[0, :, c * Skv_loc:(c + 1) * Skv_loc, :],
                    loc_sem.at[0],
                ).wait()
                pltpu.make_async_copy(
                    v_ref.at[:, :, c * Cb:(c + 1) * Cb],
                    kv_recv.at[1, :, c * Skv_loc:(c + 1) * Skv_loc, :],
                    loc_sem.at[1],
                ).wait()
        wo = wo_ref[...].astype(jnp.bfloat16)
        partials = []
        for b in range(B):
            for s in range(N_DEV):
                @pl.when(my != s)
                def _(s=s, b=b):
                    desc = pltpu.make_async_remote_copy(
                        src_ref=k_ref.at[b, :, s * Cb:(s + 1) * Cb],
                        dst_ref=kv_recv.at[0, b, pl.ds(s * Skv_loc, Skv_loc), :],
                        send_sem=kv_send_sems.at[s, b],
                        recv_sem=kv_recv_sems.at[s, b],
                        device_id=(s,),
                        device_id_type=pl.DeviceIdType.MESH,
                    )
                    desc.wait_recv()
            ws = [[] for _ in ROW_GROUPS]
            dens = [[] for _ in ROW_GROUPS]
            for h in range(Hl):
                qbh = qs[b][:, h * Dh:(h + 1) * Dh]
                for g, (r0, r1, nc) in enumerate(ROW_GROUPS):
                    kg = kv_recv[0, b, 0:nc, h * Dh:(h + 1) * Dh]
                    sc = lax.dot_general(
                        qbh[r0:r1], kg, (((1,), (1,)), ((), ())),
                        preferred_element_type=jnp.float32,
                    ) + biases[g]
                    m = sc.max(axis=1, keepdims=True)
                    w = jnp.exp(sc - m)
                    dens[g].append(w.sum(axis=1, keepdims=True))
                    ws[g].append(w.astype(jnp.bfloat16))
            for s in range(N_DEV):
                @pl.when(my != s)
                def _(s=s, b=b):
                    desc = pltpu.make_async_remote_copy(
                        src_ref=v_ref.at[b, :, s * Cb:(s + 1) * Cb],
                        dst_ref=kv_recv.at[1, b, pl.ds(s * Skv_loc, Skv_loc), :],
                        send_sem=v_send_sems.at[s, b],
                        recv_sem=v_recv_sems.at[s, b],
                        device_id=(s,),
                        device_id_type=pl.DeviceIdType.MESH,
                    )
                    desc.wait_recv()
            group_ctx = [[] for _ in ROW_GROUPS]
            for h in range(Hl):
                for g, (r0, r1, nc) in enumerate(ROW_GROUPS):
                    vg = kv_recv[1, b, 0:nc, h * Dh:(h + 1) * Dh]
                    ctx = lax.dot_general(
                        ws[g][h], vg, (((1,), (0,)), ((), ())),
                        preferred_element_type=jnp.float32,
                    ) / dens[g][h]
                    group_ctx[g].append(ctx)
            ctx_b = jnp.concatenate(
                [jnp.concatenate(hs, axis=1) for hs in group_ctx], axis=0
            ).astype(jnp.bfloat16)
            pb = lax.dot_general(
                ctx_b, wo, (((1,), (0,)), ((), ())),
                preferred_element_type=jnp.float32,
            )
            partials.append(pb)
            out_send[b] = pb.astype(jnp.bfloat16)

            for c in range(N_DEV):
                @pl.when(my != c)
                def _(c=c, b=b):
                    rdma = pltpu.make_async_remote_copy(
                        src_ref=out_send.at[b, c * Qr:(c + 1) * Qr, :],
                        dst_ref=rs_recv.at[my, b],
                        send_sem=rs_send_sems.at[c, b],
                        recv_sem=rs_recv_sems.at[my, b],
                        device_id=(c,),
                        device_id_type=pl.DeviceIdType.MESH,
                    )
                    rdma.start()

        for b in range(B):
            for s in range(N_DEV):
                @pl.when(my != s)
                def _(s=s, b=b):
                    desc = pltpu.make_async_remote_copy(
                        src_ref=out_send.at[b, s * Qr:(s + 1) * Qr, :],
                        dst_ref=rs_recv.at[s, b],
                        send_sem=rs_send_sems.at[s, b],
                        recv_sem=rs_recv_sems.at[s, b],
                        device_id=(s,),
                        device_id_type=pl.DeviceIdType.MESH,
                    )
                    desc.wait_recv()
            for c in range(N_DEV):
                @pl.when(my == c)
                def _(c=c, b=b):
                    red = partials[b][c * Qr:(c + 1) * Qr, :]
                    for s in range(N_DEV):
                        if s != c:
                            red = red + rs_recv[s, b].astype(jnp.float32)
                    out_ref[b, c * Qr:(c + 1) * Qr, :] = red
                    ag_send[b] = red.astype(jnp.bfloat16)
            for c in range(N_DEV):
                @pl.when(my != c)
                def _(c=c, b=b):
                    rdma = pltpu.make_async_remote_copy(
                        src_ref=ag_send.at[b],
                        dst_ref=ag_recv.at[my, b],
                        send_sem=ag_send_sems.at[c, b],
                        recv_sem=ag_recv_sems.at[my, b],
                        device_id=(c,),
                        device_id_type=pl.DeviceIdType.MESH,
                    )
                    rdma.start()

        for c in range(N_DEV):
            @pl.when(my != c)
            def _(c=c):
                for b in range(B):
                    pltpu.make_async_remote_copy(
                        src_ref=k_ref.at[b, :, c * Cb:(c + 1) * Cb],
                        dst_ref=kv_recv.at[0, b, pl.ds(c * Skv_loc, Skv_loc), :],
                        send_sem=kv_send_sems.at[c, b],
                        recv_sem=kv_recv_sems.at[c, b],
                        device_id=(c,),
                        device_id_type=pl.DeviceIdType.MESH,
                    ).wait_send()
                    pltpu.make_async_remote_copy(
                        src_ref=v_ref.at[b, :, c * Cb:(c + 1) * Cb],
                        dst_ref=kv_recv.at[1, b, pl.ds(c * Skv_loc, Skv_loc), :],
                        send_sem=v_send_sems.at[c, b],
                        recv_sem=v_recv_sems.at[c, b],
                        device_id=(c,),
                        device_id_type=pl.DeviceIdType.MESH,
                    ).wait_send()
                for b in range(B):
                    pltpu.make_async_remote_copy(
                        src_ref=out_send.at[b, c * Qr:(c + 1) * Qr, :],
                        dst_ref=rs_recv.at[c, b],
                        send_sem=rs_send_sems.at[c, b],
                        recv_sem=rs_recv_sems.at[c, b],
                        device_id=(c,),
                        device_id_type=pl.DeviceIdType.MESH,
                    ).wait_send()

        for b in range(B):
            for s in range(N_DEV):
                @pl.when(my != s)
                def _(s=s, b=b):
                    desc = pltpu.make_async_remote_copy(
                        src_ref=ag_send.at[b],
                        dst_ref=ag_recv.at[s, b],
                        send_sem=ag_send_sems.at[s, b],
                        recv_sem=ag_recv_sems.at[s, b],
                        device_id=(s,),
                        device_id_type=pl.DeviceIdType.MESH,
                    )
                    desc.wait_recv()
                    out_ref[b, s * Qr:(s + 1) * Qr, :] = (
                        ag_recv[s, b].astype(jnp.float32)
                    )

        for c in range(N_DEV):
            @pl.when(my != c)
            def _(c=c):
                for b in range(B):
                    pltpu.make_async_remote_copy(
                        src_ref=ag_send.at[b],
                        dst_ref=ag_recv.at[c, b],
                        send_sem=ag_send_sems.at[c, b],
                        recv_sem=ag_recv_sems.at[c, b],
                        device_id=(c,),
                        device_id_type=pl.DeviceIdType.MESH,
                    ).wait_send()

    return pl.pallas_call(
        body,
        out_shape=jax.ShapeDtypeStruct((B, Sq, Dout), jnp.float32),
        in_specs=[pl.BlockSpec(memory_space=pltpu.VMEM)] * 5,
        out_specs=pl.BlockSpec(memory_space=pltpu.VMEM),
        scratch_shapes=[
            pltpu.VMEM((2, B, N_DEV * Skv_loc, Cb), jnp.bfloat16),
            pltpu.VMEM((B, Sq, Dout), jnp.bfloat16),
            pltpu.VMEM((N_DEV, B, Qr, Dout), jnp.bfloat16),
            pltpu.VMEM((B, Qr, Dout), jnp.bfloat16),
            pltpu.VMEM((N_DEV, B, Qr, Dout), jnp.bfloat16),
            pltpu.SemaphoreType.DMA((N_DEV, B)),
            pltpu.SemaphoreType.DMA((N_DEV, B)),
            pltpu.SemaphoreType.DMA((N_DEV, B)),
            pltpu.SemaphoreType.DMA((N_DEV, B)),
            pltpu.SemaphoreType.DMA((N_DEV, B)),
            pltpu.SemaphoreType.DMA((N_DEV, B)),
            pltpu.SemaphoreType.DMA((N_DEV, B)),
            pltpu.SemaphoreType.DMA((N_DEV, B)),
            pltpu.SemaphoreType.DMA((2,)),
        ],
        compiler_params=pltpu.CompilerParams(collective_id=0),
    )(x, Wq, K2, V2, Wo)
